# baseline (speedup 1.0000x reference)
import functools

import jax
import jax.numpy as jnp
import numpy as np
from jax.experimental import pallas as pl
from jax.experimental.pallas import tpu as pltpu
from jax.experimental.pallas import tpu_sc as plsc

B = 4096
F = 26
D = 16
A = 16
P = F * (F - 1) // 2
PPAD = 384
BB = 256


def _pair_onehots():
    row = np.zeros((F, PPAD), dtype=np.float32)
    col = np.zeros((F, PPAD), dtype=np.float32)
    k = 0
    for i in range(F - 1):
        for j in range(i + 1, F):
            row[i, k] = 1.0
            col[j, k] = 1.0
            k += 1
    return row, col


KB = 16
NCHUNK = BB // KB


def _afm_body(emb_t_ref, fv_ref, fow_ref, r_ref, c_ref, pvec_ref,
              kron_ref, btile_ref, htile_ref, bias_ref, out_ref):
    fv = fv_ref[:]
    embv_t = emb_t_ref[:] * fv[:, None, :]
    x = embv_t.reshape(BB * D, F).astype(jnp.bfloat16)
    p = jnp.dot(x, r_ref[:], preferred_element_type=jnp.float32)
    q = jnp.dot(x, c_ref[:], preferred_element_type=jnp.float32)
    inter = p * q

    kron = kron_ref[:]
    btile = btile_ref[:]
    htile = htile_ref[:]
    sig_chunks = []
    for cix in range(NCHUNK):
        chunk = inter[cix * KB * D:(cix + 1) * KB * D, :].astype(jnp.bfloat16)
        t = jnp.dot(kron, chunk, preferred_element_type=jnp.float32)
        r = htile * jnp.maximum(t + btile, 0.0)
        sig_chunks.append(jnp.sum(r.reshape(KB, A, PPAD), axis=1))
    sig = jnp.concatenate(sig_chunks, axis=0)

    lane = jax.lax.broadcasted_iota(jnp.int32, (BB, PPAD), 1)
    sig = jnp.where(lane < P, sig, -1e30)
    m = jnp.max(sig, axis=1, keepdims=True)
    e = jnp.exp(sig - m)
    att = e / jnp.sum(e, axis=1, keepdims=True)

    inter3 = inter.reshape(BB, D, PPAD)
    pool = jnp.sum(att[:, None, :] * inter3, axis=2)
    yv = jnp.sum(pool * pvec_ref[:], axis=1)
    y_first = jnp.sum(fow_ref[:] * fv, axis=1)
    y = y_first + yv + bias_ref[0]
    out_ref[:] = 1.0 / (1.0 + jnp.exp(-y))


@functools.partial(jax.jit, static_argnames=())
def _afm_dense(emb_t, fv, fow, r, c, pvec, kron, btile, htile, bias):
    grid = (B // BB,)
    return pl.pallas_call(
        _afm_body,
        grid=grid,
        in_specs=[
            pl.BlockSpec((BB, D, F), lambda i: (i, 0, 0)),
            pl.BlockSpec((BB, F), lambda i: (i, 0)),
            pl.BlockSpec((BB, F), lambda i: (i, 0)),
            pl.BlockSpec((F, PPAD), lambda i: (0, 0)),
            pl.BlockSpec((F, PPAD), lambda i: (0, 0)),
            pl.BlockSpec((1, D), lambda i: (0, 0)),
            pl.BlockSpec((KB * A, KB * D), lambda i: (0, 0)),
            pl.BlockSpec((KB * A, 1), lambda i: (0, 0)),
            pl.BlockSpec((KB * A, 1), lambda i: (0, 0)),
            pl.BlockSpec(memory_space=pltpu.SMEM),
        ],
        out_specs=pl.BlockSpec((BB,), lambda i: (i,)),
        out_shape=jax.ShapeDtypeStruct((B,), jnp.float32),
    )(emb_t, fv, fow, r, c, pvec, kron, btile, htile, bias)


BF = B * F
NW = 32
PER_W = BF // NW
CH = 128
NCH = PER_W // CH


def _sc_gather_body(fi_hbm, emb_hbm, emb_out, idx_v, rows_v, sem):
    wid = jax.lax.axis_index("s") * 2 + jax.lax.axis_index("c")
    base = wid * PER_W
    pltpu.sync_copy(fi_hbm.at[pl.ds(base, PER_W)], idx_v)

    def chunk(c, carry):
        off = c * CH
        pltpu.async_copy(emb_hbm.at[idx_v.at[pl.ds(off, CH)]],
                         rows_v.at[pl.ds(off, CH)], sem).wait()
        return carry

    jax.lax.fori_loop(0, NCH, chunk, 0)
    pltpu.sync_copy(rows_v, emb_out.at[pl.ds(base, PER_W)])


def _sc_gather(fi_flat, emb_w):
    mesh = plsc.VectorSubcoreMesh(core_axis_name="c", subcore_axis_name="s")
    f = pl.kernel(
        _sc_gather_body,
        out_type=jax.ShapeDtypeStruct((BF, D), jnp.float32),
        mesh=mesh,
        scratch_types=[
            pltpu.VMEM((PER_W,), jnp.int32),
            pltpu.VMEM((PER_W, D), jnp.float32),
            pltpu.SemaphoreType.DMA,
        ],
        compiler_params=pltpu.CompilerParams(use_tc_tiling_on_sc=False),
    )
    return f(fi_flat, emb_w)


def kernel(feat_index, feat_value, fo_w, emb_w, att_W, att_b, att_h, p_vec, bias):
    fi = feat_index.astype(jnp.int32)
    emb_rows = _sc_gather(fi.reshape(BF), emb_w)
    emb_t = emb_rows.reshape(B, F, D).transpose(0, 2, 1)
    fow = jnp.take(fo_w[:, 0], fi, axis=0)
    r_np, c_np = _pair_onehots()
    r = jnp.asarray(r_np, dtype=jnp.bfloat16)
    c = jnp.asarray(c_np, dtype=jnp.bfloat16)
    pvec = p_vec.reshape(1, D)
    kron = jnp.kron(jnp.eye(KB, dtype=jnp.float32), att_W.T).astype(jnp.bfloat16)
    btile = jnp.tile(att_b, KB).reshape(KB * A, 1)
    htile = jnp.tile(att_h, KB).reshape(KB * A, 1)
    return _afm_dense(emb_t, feat_value, fow, r, c, pvec,
                      kron, btile, htile, bias)


# speedup vs baseline: 1.0987x; 1.0987x over previous
import functools

import jax
import jax.numpy as jnp
import numpy as np
from jax.experimental import pallas as pl
from jax.experimental.pallas import tpu as pltpu
from jax.experimental.pallas import tpu_sc as plsc

B = 4096
F = 26
D = 16
A = 16
P = F * (F - 1) // 2
PPAD = 384
BB = 256


def _pair_onehots():
    row = np.zeros((F, PPAD), dtype=np.float32)
    col = np.zeros((F, PPAD), dtype=np.float32)
    k = 0
    for i in range(F - 1):
        for j in range(i + 1, F):
            row[i, k] = 1.0
            col[j, k] = 1.0
            k += 1
    return row, col


KB = 16
NCHUNK = BB // KB


def _afm_body(emb_t_ref, fv_ref, fow_ref, r_ref, c_ref, pvec_ref,
              kron_ref, btile_ref, htile_ref, bias_ref, out_ref):
    fv = fv_ref[:]
    embv_t = emb_t_ref[:] * fv[:, None, :]
    x = embv_t.reshape(BB * D, F).astype(jnp.bfloat16)
    p = jnp.dot(x, r_ref[:], preferred_element_type=jnp.float32)
    q = jnp.dot(x, c_ref[:], preferred_element_type=jnp.float32)
    inter = p * q

    kron = kron_ref[:]
    btile = btile_ref[:]
    htile = htile_ref[:]
    sig_chunks = []
    for cix in range(NCHUNK):
        chunk = inter[cix * KB * D:(cix + 1) * KB * D, :].astype(jnp.bfloat16)
        t = jnp.dot(kron, chunk, preferred_element_type=jnp.float32)
        r = htile * jnp.maximum(t + btile, 0.0)
        sig_chunks.append(jnp.sum(r.reshape(KB, A, PPAD), axis=1))
    sig = jnp.concatenate(sig_chunks, axis=0)

    lane = jax.lax.broadcasted_iota(jnp.int32, (BB, PPAD), 1)
    sig = jnp.where(lane < P, sig, -1e30)
    m = jnp.max(sig, axis=1, keepdims=True)
    e = jnp.exp(sig - m)
    att = e / jnp.sum(e, axis=1, keepdims=True)

    inter3 = inter.reshape(BB, D, PPAD)
    pool = jnp.sum(att[:, None, :] * inter3, axis=2)
    yv = jnp.sum(pool * pvec_ref[:], axis=1)
    y_first = jnp.sum(fow_ref[:] * fv, axis=1)
    y = y_first + yv + bias_ref[0]
    out_ref[:] = 1.0 / (1.0 + jnp.exp(-y))


@functools.partial(jax.jit, static_argnames=())
def _afm_dense(emb_t, fv, fow, r, c, pvec, kron, btile, htile, bias):
    grid = (B // BB,)
    return pl.pallas_call(
        _afm_body,
        grid=grid,
        in_specs=[
            pl.BlockSpec((BB, D, F), lambda i: (i, 0, 0)),
            pl.BlockSpec((BB, F), lambda i: (i, 0)),
            pl.BlockSpec((BB, F), lambda i: (i, 0)),
            pl.BlockSpec((F, PPAD), lambda i: (0, 0)),
            pl.BlockSpec((F, PPAD), lambda i: (0, 0)),
            pl.BlockSpec((1, D), lambda i: (0, 0)),
            pl.BlockSpec((KB * A, KB * D), lambda i: (0, 0)),
            pl.BlockSpec((KB * A, 1), lambda i: (0, 0)),
            pl.BlockSpec((KB * A, 1), lambda i: (0, 0)),
            pl.BlockSpec(memory_space=pltpu.SMEM),
        ],
        out_specs=pl.BlockSpec((BB,), lambda i: (i,)),
        out_shape=jax.ShapeDtypeStruct((B,), jnp.float32),
    )(emb_t, fv, fow, r, c, pvec, kron, btile, htile, bias)


NUM_FEATS = 1000000
BF = B * F
NW = 32
PER_W = BF // NW
CH = 128
NCH = PER_W // CH


GRP = 8


def _sc_gather_body(fi_hbm, emb_hbm, emb_out, idx_v, hi_v, blk_v, rows_v, sem):
    wid = jax.lax.axis_index("s") * 2 + jax.lax.axis_index("c")
    base = wid * PER_W
    pltpu.sync_copy(fi_hbm.at[pl.ds(base, PER_W)], idx_v)

    def grp16(g, carry):
        hi_v[pl.ds(g * 16, 16)] = jax.lax.shift_right_logical(
            idx_v[pl.ds(g * 16, 16)], 3)
        return carry

    jax.lax.fori_loop(0, PER_W // 16, grp16, 0)

    def chunk(c, carry):
        off = c * CH
        pltpu.async_copy(emb_hbm.at[hi_v.at[pl.ds(off, CH)]],
                         blk_v, sem).wait()

        def extract(g, carry2):
            lo16 = idx_v[pl.ds(off + g * 16, 16)] & 7
            for j in range(16):
                s = lo16[j]
                k = g * 16 + j
                rows_v[pl.ds((off + k) * D, D)] = (
                    blk_v[k, pl.ds(s * D, D)])
            return carry2

        jax.lax.fori_loop(0, CH // 16, extract, 0)
        return carry

    jax.lax.fori_loop(0, NCH, chunk, 0)
    pltpu.sync_copy(rows_v, emb_out.at[pl.ds(base * D, PER_W * D)])


def _sc_gather(fi_flat, emb_grp):
    mesh = plsc.VectorSubcoreMesh(core_axis_name="c", subcore_axis_name="s")
    f = pl.kernel(
        _sc_gather_body,
        out_type=jax.ShapeDtypeStruct((BF * D,), jnp.float32),
        mesh=mesh,
        scratch_types=[
            pltpu.VMEM((PER_W,), jnp.int32),
            pltpu.VMEM((PER_W,), jnp.int32),
            pltpu.VMEM((CH, GRP * D), jnp.float32),
            pltpu.VMEM((PER_W * D,), jnp.float32),
            pltpu.SemaphoreType.DMA,
        ],
    )
    return f(fi_flat, emb_grp)


def kernel(feat_index, feat_value, fo_w, emb_w, att_W, att_b, att_h, p_vec, bias):
    fi = feat_index.astype(jnp.int32)
    emb_grp = emb_w.reshape(NUM_FEATS // GRP, GRP * D)
    emb_rows = _sc_gather(fi.reshape(BF), emb_grp)
    emb_t = emb_rows.reshape(B, F, D).transpose(0, 2, 1)
    fow = jnp.take(fo_w[:, 0], fi, axis=0)
    r_np, c_np = _pair_onehots()
    r = jnp.asarray(r_np, dtype=jnp.bfloat16)
    c = jnp.asarray(c_np, dtype=jnp.bfloat16)
    pvec = p_vec.reshape(1, D)
    kron = jnp.kron(jnp.eye(KB, dtype=jnp.float32), att_W.T).astype(jnp.bfloat16)
    btile = jnp.tile(att_b, KB).reshape(KB * A, 1)
    htile = jnp.tile(att_h, KB).reshape(KB * A, 1)
    return _afm_dense(emb_t, feat_value, fow, r, c, pvec,
                      kron, btile, htile, bias)


# speedup vs baseline: 2.2406x; 2.0393x over previous
import functools

import jax
import jax.numpy as jnp
import numpy as np
from jax.experimental import pallas as pl
from jax.experimental.pallas import tpu as pltpu

B = 4096
F = 26
D = 16
A = 16
P = F * (F - 1) // 2
PPAD = 384
BB = 256


def _pair_onehots():
    row = np.zeros((F, PPAD), dtype=np.float32)
    col = np.zeros((F, PPAD), dtype=np.float32)
    k = 0
    for i in range(F - 1):
        for j in range(i + 1, F):
            row[i, k] = 1.0
            col[j, k] = 1.0
            k += 1
    return row, col


KB = 16
NCHUNK = BB // KB


def _afm_body(emb_t_ref, fv_ref, fow_ref, r_ref, c_ref, pvec_ref,
              kron_ref, btile_ref, htile_ref, bias_ref, out_ref):
    fv = fv_ref[:]
    embv_t = emb_t_ref[:] * fv[:, None, :]
    x = embv_t.reshape(BB * D, F).astype(jnp.bfloat16)
    p = jnp.dot(x, r_ref[:], preferred_element_type=jnp.float32)
    q = jnp.dot(x, c_ref[:], preferred_element_type=jnp.float32)
    inter = p * q

    kron = kron_ref[:]
    btile = btile_ref[:]
    htile = htile_ref[:]
    sig_chunks = []
    for cix in range(NCHUNK):
        chunk = inter[cix * KB * D:(cix + 1) * KB * D, :].astype(jnp.bfloat16)
        t = jnp.dot(kron, chunk, preferred_element_type=jnp.float32)
        r = htile * jnp.maximum(t + btile, 0.0)
        sig_chunks.append(jnp.sum(r.reshape(KB, A, PPAD), axis=1))
    sig = jnp.concatenate(sig_chunks, axis=0)

    lane = jax.lax.broadcasted_iota(jnp.int32, (BB, PPAD), 1)
    sig = jnp.where(lane < P, sig, -1e30)
    m = jnp.max(sig, axis=1, keepdims=True)
    e = jnp.exp(sig - m)
    att = e / jnp.sum(e, axis=1, keepdims=True)

    inter3 = inter.reshape(BB, D, PPAD)
    pool = jnp.sum(att[:, None, :] * inter3, axis=2)
    yv = jnp.sum(pool * pvec_ref[:], axis=1)
    y_first = jnp.sum(fow_ref[:] * fv, axis=1)
    y = y_first + yv + bias_ref[0]
    out_ref[:] = 1.0 / (1.0 + jnp.exp(-y))


@functools.partial(jax.jit, static_argnames=())
def _afm_dense(emb_t, fv, fow, r, c, pvec, kron, btile, htile, bias):
    grid = (B // BB,)
    return pl.pallas_call(
        _afm_body,
        grid=grid,
        in_specs=[
            pl.BlockSpec((BB, D, F), lambda i: (i, 0, 0)),
            pl.BlockSpec((BB, F), lambda i: (i, 0)),
            pl.BlockSpec((BB, F), lambda i: (i, 0)),
            pl.BlockSpec((F, PPAD), lambda i: (0, 0)),
            pl.BlockSpec((F, PPAD), lambda i: (0, 0)),
            pl.BlockSpec((1, D), lambda i: (0, 0)),
            pl.BlockSpec((KB * A, KB * D), lambda i: (0, 0)),
            pl.BlockSpec((KB * A, 1), lambda i: (0, 0)),
            pl.BlockSpec((KB * A, 1), lambda i: (0, 0)),
            pl.BlockSpec(memory_space=pltpu.SMEM),
        ],
        out_specs=pl.BlockSpec((BB,), lambda i: (i,)),
        out_shape=jax.ShapeDtypeStruct((B,), jnp.float32),
    )(emb_t, fv, fow, r, c, pvec, kron, btile, htile, bias)


def kernel(feat_index, feat_value, fo_w, emb_w, att_W, att_b, att_h, p_vec, bias):
    fi = feat_index.astype(jnp.int32)
    emb = jnp.take(emb_w, fi, axis=0)
    emb_t = emb.transpose(0, 2, 1)
    fow = jnp.take(fo_w[:, 0], fi, axis=0)
    r_np, c_np = _pair_onehots()
    r = jnp.asarray(r_np, dtype=jnp.bfloat16)
    c = jnp.asarray(c_np, dtype=jnp.bfloat16)
    pvec = p_vec.reshape(1, D)
    kron = jnp.kron(jnp.eye(KB, dtype=jnp.float32), att_W.T).astype(jnp.bfloat16)
    btile = jnp.tile(att_b, KB).reshape(KB * A, 1)
    htile = jnp.tile(att_h, KB).reshape(KB * A, 1)
    return _afm_dense(emb_t, feat_value, fow, r, c, pvec,
                      kron, btile, htile, bias)
